# Initial kernel scaffold; baseline (speedup 1.0000x reference)
import jax
import jax.numpy as jnp
from jax.experimental import pallas as pl


def kernel(x, edge_index, params):
    raise NotImplementedError("write your pallas kernel here")


# speedup vs baseline: 3.5839x; 3.5839x over previous
import functools

import jax
import jax.numpy as jnp
from jax import lax
from jax.experimental import pallas as pl
from jax.experimental.pallas import tpu as pltpu
from jax.experimental.pallas import tpu_sc as plsc

N = 10000
E = 320000
NC = 2
NS = 16
K = 80
M_BLK = 1000
GRID = N // M_BLK
BN_EPS = 1e-5


def _sc_segment_sum(t0, t1, src, dst, zeros, mode):
    if mode == 'col':
        epw = E // NS
    else:
        epw = E // (NS * NC)
    chunks = epw // K
    rows_per_sub = 632

    mesh = plsc.VectorSubcoreMesh(core_axis_name="c", subcore_axis_name="s")

    @functools.partial(
        pl.kernel,
        out_type=(jax.ShapeDtypeStruct((N, 128), jnp.float32),
                  jax.ShapeDtypeStruct((N, 128), jnp.float32)),
        mesh=mesh,
        scratch_types=[
            pltpu.VMEM((K,), jnp.int32),
            pltpu.VMEM((K,), jnp.int32),
            pltpu.VMEM((K, 128), jnp.float32),
            pltpu.VMEM_SHARED((N, 128), jnp.float32),
            pltpu.SemaphoreType.DMA,
        ],
    )
    def kern(t0_h, t1_h, src_h, dst_h, zero_h, out0_h, out1_h,
             src_v, dst_v, rows_v, acc, sem):
        c = lax.axis_index("c")
        s = lax.axis_index("s")
        r0 = jnp.minimum(s * rows_per_sub, N - rows_per_sub)
        pltpu.sync_copy(zero_h.at[pl.ds(r0, rows_per_sub)],
                        acc.at[pl.ds(r0, rows_per_sub)])
        plsc.subcore_barrier()

        def run(table_h, out_h, base0):
            def body(i, carry):
                b = base0 + i * K
                pltpu.sync_copy(src_h.at[pl.ds(b, K)], src_v)
                pltpu.sync_copy(dst_h.at[pl.ds(b, K)], dst_v)
                pltpu.async_copy(table_h.at[src_v], rows_v, sem).wait()
                pltpu.sync_copy(rows_v, acc.at[dst_v], add=True)
                return carry
            lax.fori_loop(0, chunks, body, 0)
            plsc.subcore_barrier()
            pltpu.sync_copy(acc.at[pl.ds(r0, rows_per_sub)],
                            out_h.at[pl.ds(r0, rows_per_sub)])

        if mode == 'col':
            base0 = s * epw
            base1 = s * epw
        else:
            base0 = (s * NC + 0) * epw
            base1 = (s * NC + 1) * epw

        @pl.when(c == 0)
        def _():
            run(t0_h, out0_h, base0)

        @pl.when(c == 1)
        def _():
            run(t1_h, out1_h, base1)

    return kern(t0, t1, src, dst, zeros)


def _row_spec(cols):
    return pl.BlockSpec((M_BLK, cols), lambda i: (i, 0))


def _fix_spec(shape):
    return pl.BlockSpec(shape, lambda i: (0, 0))


def _emit_stats(i, h, st_ref):
    @pl.when(i == 0)
    def _():
        st_ref[...] = jnp.zeros_like(st_ref)
    st_ref[0:1, :] = st_ref[0:1, :] + jnp.sum(h, axis=0, keepdims=True)
    st_ref[1:2, :] = st_ref[1:2, :] + jnp.sum(h * h, axis=0, keepdims=True)


def _normalize(h, st_ref, g, be, relu):
    mu = st_ref[0:1, :] * (1.0 / N)
    var = st_ref[1:2, :] * (1.0 / N) - mu * mu
    out = g * (h - mu) * lax.rsqrt(var + BN_EPS) + be
    if relu:
        out = jnp.maximum(out, 0.0)
    return out


def _p1_first(x, p0, p1, scale, W1, b1):
    def kern(scale_ref, x_ref, p0_ref, p1_ref, W_ref, b_ref, h_ref, st_ref):
        i = pl.program_id(0)
        z = scale_ref[0, 0] * x_ref[...] + p0_ref[...] + p1_ref[...]
        h = jnp.dot(z, W_ref[...], preferred_element_type=jnp.float32) + b_ref[...]
        h_ref[...] = h
        _emit_stats(i, h, st_ref)

    return pl.pallas_call(
        kern,
        grid=(GRID,),
        in_specs=[pl.BlockSpec(memory_space=pltpu.SMEM),
                  _row_spec(128), _row_spec(128), _row_spec(128),
                  _fix_spec((128, 256)), _fix_spec((1, 256))],
        out_specs=[_row_spec(256), _fix_spec((8, 256))],
        out_shape=[jax.ShapeDtypeStruct((N, 256), jnp.float32),
                   jax.ShapeDtypeStruct((8, 256), jnp.float32)],
    )(scale, x, p0, p1, W1, b1)


def _p1_split(h0, h1c, a0, a1, scale, W1, b1):
    def kern(scale_ref, h0_ref, h1_ref, a0_ref, a1_ref, W_ref, b_ref,
             h_ref, st_ref):
        i = pl.program_id(0)
        sc = scale_ref[0, 0]
        z0 = sc * h0_ref[...] + a0_ref[...]
        z1 = sc * h1_ref[...] + a1_ref[...]
        h = (jnp.dot(z0, W_ref[0:128, :], preferred_element_type=jnp.float32)
             + jnp.dot(z1, W_ref[128:256, :], preferred_element_type=jnp.float32)
             + b_ref[...])
        h_ref[...] = h
        _emit_stats(i, h, st_ref)

    return pl.pallas_call(
        kern,
        grid=(GRID,),
        in_specs=[pl.BlockSpec(memory_space=pltpu.SMEM),
                  _row_spec(128), _row_spec(128), _row_spec(128), _row_spec(128),
                  _fix_spec((256, 256)), _fix_spec((1, 256))],
        out_specs=[_row_spec(256), _fix_spec((8, 256))],
        out_shape=[jax.ShapeDtypeStruct((N, 256), jnp.float32),
                   jax.ShapeDtypeStruct((8, 256), jnp.float32)],
    )(scale, h0, h1c, a0, a1, W1, b1)


def _norm_mm(h_in, st, g, be, W, b):
    def kern(h_ref, st_in_ref, g_ref, be_ref, W_ref, b_ref, h_ref_o, st_ref):
        i = pl.program_id(0)
        a = _normalize(h_ref[...], st_in_ref, g_ref[...], be_ref[...], relu=True)
        h = jnp.dot(a, W_ref[...], preferred_element_type=jnp.float32) + b_ref[...]
        h_ref_o[...] = h
        _emit_stats(i, h, st_ref)

    return pl.pallas_call(
        kern,
        grid=(GRID,),
        in_specs=[_row_spec(256), _fix_spec((8, 256)),
                  _fix_spec((1, 256)), _fix_spec((1, 256)),
                  _fix_spec((256, 256)), _fix_spec((1, 256))],
        out_specs=[_row_spec(256), _fix_spec((8, 256))],
        out_shape=[jax.ShapeDtypeStruct((N, 256), jnp.float32),
                   jax.ShapeDtypeStruct((8, 256), jnp.float32)],
    )(h_in, st, g, be, W, b)


def _norm_split(h_in, st, g, b):
    def kern(h_ref, st_in_ref, g_ref, b_ref, o0_ref, o1_ref):
        h = _normalize(h_ref[...], st_in_ref, g_ref[...], b_ref[...], relu=False)
        o0_ref[...] = h[:, 0:128]
        o1_ref[...] = h[:, 128:256]

    return pl.pallas_call(
        kern,
        grid=(GRID,),
        in_specs=[_row_spec(256), _fix_spec((8, 256)),
                  _fix_spec((1, 256)), _fix_spec((1, 256))],
        out_specs=[_row_spec(128), _row_spec(128)],
        out_shape=[jax.ShapeDtypeStruct((N, 128), jnp.float32),
                   jax.ShapeDtypeStruct((N, 128), jnp.float32)],
    )(h_in, st, g, b)


def _head(ha0, ha1, hb0, hb1, hc, Wf1, bf1, Wf2, bf2):
    def kern(ha0_ref, ha1_ref, hb0_ref, hb1_ref, hc_ref, W1_ref, b1_ref,
             W2_ref, b2_ref, o_ref):
        t = (jnp.dot(ha0_ref[...], W1_ref[0:128, :], preferred_element_type=jnp.float32)
             + jnp.dot(ha1_ref[...], W1_ref[128:256, :], preferred_element_type=jnp.float32)
             + jnp.dot(hb0_ref[...], W1_ref[256:384, :], preferred_element_type=jnp.float32)
             + jnp.dot(hb1_ref[...], W1_ref[384:512, :], preferred_element_type=jnp.float32)
             + jnp.dot(hc_ref[...], W1_ref[512:768, :], preferred_element_type=jnp.float32)
             + b1_ref[...])
        t = jnp.maximum(t, 0.0)
        o_ref[...] = jnp.dot(t, W2_ref[...], preferred_element_type=jnp.float32) + b2_ref[...]

    return pl.pallas_call(
        kern,
        grid=(GRID,),
        in_specs=[_row_spec(128), _row_spec(128), _row_spec(128), _row_spec(128),
                  _row_spec(256),
                  _fix_spec((768, 256)), _fix_spec((1, 256)),
                  _fix_spec((256, 128)), _fix_spec((1, 128))],
        out_specs=pl.BlockSpec((M_BLK, 128), lambda i: (i, 0)),
        out_shape=jax.ShapeDtypeStruct((N, 128), jnp.float32),
    )(ha0, ha1, hb0, hb1, hc, Wf1, bf1, Wf2, bf2)


def kernel(x, edge_index, params):
    src = edge_index[0].astype(jnp.int32)
    dst = edge_index[1].astype(jnp.int32)
    zeros = jnp.zeros((N, 128), jnp.float32)
    row = lambda v: v.reshape(1, -1)

    convs = params['convs']
    bns = params['bns']

    def scale_of(p):
        return (1.0 + p['eps']).reshape(1, 1)

    p0, p1 = _sc_segment_sum(x, x, src, dst, zeros, mode='edge')
    c0 = convs[0]
    h, st = _p1_first(x, p0, p1, scale_of(c0), c0['W1'], row(c0['b1']))
    h, st = _norm_mm(h, st, row(c0['g1']), row(c0['be1']), c0['W2'], row(c0['b2']))
    h, st = _norm_mm(h, st, row(c0['g2']), row(c0['be2']), c0['W3'], row(c0['b3']))
    ha0, ha1 = _norm_split(h, st, row(bns[0]['g']), row(bns[0]['b']))

    a0, a1 = _sc_segment_sum(ha0, ha1, src, dst, zeros, mode='col')
    c1 = convs[1]
    h, st = _p1_split(ha0, ha1, a0, a1, scale_of(c1), c1['W1'], row(c1['b1']))
    h, st = _norm_mm(h, st, row(c1['g1']), row(c1['be1']), c1['W2'], row(c1['b2']))
    h, st = _norm_mm(h, st, row(c1['g2']), row(c1['be2']), c1['W3'], row(c1['b3']))
    hb0, hb1 = _norm_split(h, st, row(bns[1]['g']), row(bns[1]['b']))

    a0, a1 = _sc_segment_sum(hb0, hb1, src, dst, zeros, mode='col')
    c2 = convs[2]
    h, st = _p1_split(hb0, hb1, a0, a1, scale_of(c2), c2['W1'], row(c2['b1']))
    h, st = _norm_mm(h, st, row(c2['g1']), row(c2['be1']), c2['W2'], row(c2['b2']))
    hc, _ = _norm_mm(h, st, row(c2['g2']), row(c2['be2']), c2['W3'], row(c2['b3']))

    return _head(ha0, ha1, hb0, hb1, hc,
                 params['fc1']['W'], row(params['fc1']['b']),
                 params['fc2']['W'], row(params['fc2']['b']))


# speedup vs baseline: 5.1647x; 1.4411x over previous
import functools

import jax
import jax.numpy as jnp
from jax import lax
from jax.experimental import pallas as pl
from jax.experimental.pallas import tpu as pltpu
from jax.experimental.pallas import tpu_sc as plsc

N = 10000
E = 320000
NC = 2
NS = 16
K = 80
M_BLK = 1000
GRID = N // M_BLK
BN_EPS = 1e-5


def _sc_segment_sum(t0, t1, src, dst, zeros, mode):
    if mode == 'col':
        epw = E // NS
        k = K
    else:
        epw = E // (NS * NC)
        k = K // 2
    chunks = epw // k
    rows_per_sub = 632

    mesh = plsc.VectorSubcoreMesh(core_axis_name="c", subcore_axis_name="s")

    @functools.partial(
        pl.kernel,
        out_type=(jax.ShapeDtypeStruct((N, 128), jnp.float32),
                  jax.ShapeDtypeStruct((N, 128), jnp.float32)),
        mesh=mesh,
        scratch_types=[
            pltpu.VMEM((k,), jnp.int32),
            pltpu.VMEM((k,), jnp.int32),
            pltpu.VMEM((k, 128), jnp.float32),
            pltpu.VMEM((k,), jnp.int32),
            pltpu.VMEM((k,), jnp.int32),
            pltpu.VMEM((k, 128), jnp.float32),
            pltpu.VMEM_SHARED((N, 128), jnp.float32),
            pltpu.SemaphoreType.DMA,
            pltpu.SemaphoreType.DMA,
        ],
    )
    def kern(t0_h, t1_h, src_h, dst_h, zero_h, out0_h, out1_h,
             src_v0, dst_v0, rows_v0, src_v1, dst_v1, rows_v1, acc,
             sem0, sem1):
        bufs = ((src_v0, dst_v0, rows_v0, sem0),
                (src_v1, dst_v1, rows_v1, sem1))
        c = lax.axis_index("c")
        s = lax.axis_index("s")
        r0 = jnp.minimum(s * rows_per_sub, N - rows_per_sub)
        pltpu.sync_copy(zero_h.at[pl.ds(r0, rows_per_sub)],
                        acc.at[pl.ds(r0, rows_per_sub)])
        plsc.subcore_barrier()

        def run(table_h, out_h, base0):
            def prefetch(i, buf):
                sv, dv, rv, sm = buf
                b = base0 + i * k
                pltpu.sync_copy(src_h.at[pl.ds(b, k)], sv)
                pltpu.sync_copy(dst_h.at[pl.ds(b, k)], dv)
                pltpu.async_copy(table_h.at[sv], rv, sm)

            prefetch(0, bufs[0])
            prefetch(1, bufs[1])

            @pl.loop(0, chunks, step=2)
            def _(g):
                for b in (0, 1):
                    sv, dv, rv, sm = bufs[b]
                    i = g + b
                    pltpu.make_async_copy(table_h.at[sv], rv, sm).wait()
                    pltpu.sync_copy(rv, acc.at[dv], add=True)

                    @pl.when(i + 2 < chunks)
                    def _():
                        prefetch(i + 2, bufs[b])

            plsc.subcore_barrier()
            pltpu.sync_copy(acc.at[pl.ds(r0, rows_per_sub)],
                            out_h.at[pl.ds(r0, rows_per_sub)])

        if mode == 'col':
            base0 = s * epw
            base1 = s * epw
        else:
            base0 = (s * NC + 0) * epw
            base1 = (s * NC + 1) * epw

        @pl.when(c == 0)
        def _():
            run(t0_h, out0_h, base0)

        @pl.when(c == 1)
        def _():
            run(t1_h, out1_h, base1)

    return kern(t0, t1, src, dst, zeros)


def _row_spec(cols):
    return pl.BlockSpec((M_BLK, cols), lambda i: (i, 0))


def _fix_spec(shape):
    return pl.BlockSpec(shape, lambda i: (0, 0))


def _emit_stats(i, h, st_ref):
    @pl.when(i == 0)
    def _():
        st_ref[...] = jnp.zeros_like(st_ref)
    st_ref[0:1, :] = st_ref[0:1, :] + jnp.sum(h, axis=0, keepdims=True)
    st_ref[1:2, :] = st_ref[1:2, :] + jnp.sum(h * h, axis=0, keepdims=True)


def _normalize(h, st_ref, g, be, relu):
    mu = st_ref[0:1, :] * (1.0 / N)
    var = st_ref[1:2, :] * (1.0 / N) - mu * mu
    out = g * (h - mu) * lax.rsqrt(var + BN_EPS) + be
    if relu:
        out = jnp.maximum(out, 0.0)
    return out


def _p1_first(x, p0, p1, scale, W1, b1):
    def kern(scale_ref, x_ref, p0_ref, p1_ref, W_ref, b_ref, h_ref, st_ref):
        i = pl.program_id(0)
        z = scale_ref[0, 0] * x_ref[...] + p0_ref[...] + p1_ref[...]
        h = jnp.dot(z, W_ref[...], preferred_element_type=jnp.float32) + b_ref[...]
        h_ref[...] = h
        _emit_stats(i, h, st_ref)

    return pl.pallas_call(
        kern,
        grid=(GRID,),
        in_specs=[pl.BlockSpec(memory_space=pltpu.SMEM),
                  _row_spec(128), _row_spec(128), _row_spec(128),
                  _fix_spec((128, 256)), _fix_spec((1, 256))],
        out_specs=[_row_spec(256), _fix_spec((8, 256))],
        out_shape=[jax.ShapeDtypeStruct((N, 256), jnp.float32),
                   jax.ShapeDtypeStruct((8, 256), jnp.float32)],
    )(scale, x, p0, p1, W1, b1)


def _p1_split(h0, h1c, a0, a1, scale, W1, b1):
    def kern(scale_ref, h0_ref, h1_ref, a0_ref, a1_ref, W_ref, b_ref,
             h_ref, st_ref):
        i = pl.program_id(0)
        sc = scale_ref[0, 0]
        z0 = sc * h0_ref[...] + a0_ref[...]
        z1 = sc * h1_ref[...] + a1_ref[...]
        h = (jnp.dot(z0, W_ref[0:128, :], preferred_element_type=jnp.float32)
             + jnp.dot(z1, W_ref[128:256, :], preferred_element_type=jnp.float32)
             + b_ref[...])
        h_ref[...] = h
        _emit_stats(i, h, st_ref)

    return pl.pallas_call(
        kern,
        grid=(GRID,),
        in_specs=[pl.BlockSpec(memory_space=pltpu.SMEM),
                  _row_spec(128), _row_spec(128), _row_spec(128), _row_spec(128),
                  _fix_spec((256, 256)), _fix_spec((1, 256))],
        out_specs=[_row_spec(256), _fix_spec((8, 256))],
        out_shape=[jax.ShapeDtypeStruct((N, 256), jnp.float32),
                   jax.ShapeDtypeStruct((8, 256), jnp.float32)],
    )(scale, h0, h1c, a0, a1, W1, b1)


def _norm_mm(h_in, st, g, be, W, b):
    def kern(h_ref, st_in_ref, g_ref, be_ref, W_ref, b_ref, h_ref_o, st_ref):
        i = pl.program_id(0)
        a = _normalize(h_ref[...], st_in_ref, g_ref[...], be_ref[...], relu=True)
        h = jnp.dot(a, W_ref[...], preferred_element_type=jnp.float32) + b_ref[...]
        h_ref_o[...] = h
        _emit_stats(i, h, st_ref)

    return pl.pallas_call(
        kern,
        grid=(GRID,),
        in_specs=[_row_spec(256), _fix_spec((8, 256)),
                  _fix_spec((1, 256)), _fix_spec((1, 256)),
                  _fix_spec((256, 256)), _fix_spec((1, 256))],
        out_specs=[_row_spec(256), _fix_spec((8, 256))],
        out_shape=[jax.ShapeDtypeStruct((N, 256), jnp.float32),
                   jax.ShapeDtypeStruct((8, 256), jnp.float32)],
    )(h_in, st, g, be, W, b)


def _norm_split(h_in, st, g, b):
    def kern(h_ref, st_in_ref, g_ref, b_ref, o0_ref, o1_ref):
        h = _normalize(h_ref[...], st_in_ref, g_ref[...], b_ref[...], relu=False)
        o0_ref[...] = h[:, 0:128]
        o1_ref[...] = h[:, 128:256]

    return pl.pallas_call(
        kern,
        grid=(GRID,),
        in_specs=[_row_spec(256), _fix_spec((8, 256)),
                  _fix_spec((1, 256)), _fix_spec((1, 256))],
        out_specs=[_row_spec(128), _row_spec(128)],
        out_shape=[jax.ShapeDtypeStruct((N, 128), jnp.float32),
                   jax.ShapeDtypeStruct((N, 128), jnp.float32)],
    )(h_in, st, g, b)


def _head(ha0, ha1, hb0, hb1, hc, Wf1, bf1, Wf2, bf2):
    def kern(ha0_ref, ha1_ref, hb0_ref, hb1_ref, hc_ref, W1_ref, b1_ref,
             W2_ref, b2_ref, o_ref):
        t = (jnp.dot(ha0_ref[...], W1_ref[0:128, :], preferred_element_type=jnp.float32)
             + jnp.dot(ha1_ref[...], W1_ref[128:256, :], preferred_element_type=jnp.float32)
             + jnp.dot(hb0_ref[...], W1_ref[256:384, :], preferred_element_type=jnp.float32)
             + jnp.dot(hb1_ref[...], W1_ref[384:512, :], preferred_element_type=jnp.float32)
             + jnp.dot(hc_ref[...], W1_ref[512:768, :], preferred_element_type=jnp.float32)
             + b1_ref[...])
        t = jnp.maximum(t, 0.0)
        o_ref[...] = jnp.dot(t, W2_ref[...], preferred_element_type=jnp.float32) + b2_ref[...]

    return pl.pallas_call(
        kern,
        grid=(GRID,),
        in_specs=[_row_spec(128), _row_spec(128), _row_spec(128), _row_spec(128),
                  _row_spec(256),
                  _fix_spec((768, 256)), _fix_spec((1, 256)),
                  _fix_spec((256, 128)), _fix_spec((1, 128))],
        out_specs=pl.BlockSpec((M_BLK, 128), lambda i: (i, 0)),
        out_shape=jax.ShapeDtypeStruct((N, 128), jnp.float32),
    )(ha0, ha1, hb0, hb1, hc, Wf1, bf1, Wf2, bf2)


def kernel(x, edge_index, params):
    src = edge_index[0].astype(jnp.int32)
    dst = edge_index[1].astype(jnp.int32)
    zeros = jnp.zeros((N, 128), jnp.float32)
    row = lambda v: v.reshape(1, -1)

    convs = params['convs']
    bns = params['bns']

    def scale_of(p):
        return (1.0 + p['eps']).reshape(1, 1)

    p0, p1 = _sc_segment_sum(x, x, src, dst, zeros, mode='edge')
    c0 = convs[0]
    h, st = _p1_first(x, p0, p1, scale_of(c0), c0['W1'], row(c0['b1']))
    h, st = _norm_mm(h, st, row(c0['g1']), row(c0['be1']), c0['W2'], row(c0['b2']))
    h, st = _norm_mm(h, st, row(c0['g2']), row(c0['be2']), c0['W3'], row(c0['b3']))
    ha0, ha1 = _norm_split(h, st, row(bns[0]['g']), row(bns[0]['b']))

    a0, a1 = _sc_segment_sum(ha0, ha1, src, dst, zeros, mode='col')
    c1 = convs[1]
    h, st = _p1_split(ha0, ha1, a0, a1, scale_of(c1), c1['W1'], row(c1['b1']))
    h, st = _norm_mm(h, st, row(c1['g1']), row(c1['be1']), c1['W2'], row(c1['b2']))
    h, st = _norm_mm(h, st, row(c1['g2']), row(c1['be2']), c1['W3'], row(c1['b3']))
    hb0, hb1 = _norm_split(h, st, row(bns[1]['g']), row(bns[1]['b']))

    a0, a1 = _sc_segment_sum(hb0, hb1, src, dst, zeros, mode='col')
    c2 = convs[2]
    h, st = _p1_split(hb0, hb1, a0, a1, scale_of(c2), c2['W1'], row(c2['b1']))
    h, st = _norm_mm(h, st, row(c2['g1']), row(c2['be1']), c2['W2'], row(c2['b2']))
    hc, _ = _norm_mm(h, st, row(c2['g2']), row(c2['be2']), c2['W3'], row(c2['b3']))

    return _head(ha0, ha1, hb0, hb1, hc,
                 params['fc1']['W'], row(params['fc1']['b']),
                 params['fc2']['W'], row(params['fc2']['b']))


# speedup vs baseline: 9.2864x; 1.7980x over previous
import functools

import jax
import jax.numpy as jnp
from jax import lax
from jax.experimental import pallas as pl
from jax.experimental.pallas import tpu as pltpu
from jax.experimental.pallas import tpu_sc as plsc

N = 10000
E = 320000
NC = 2
NS = 16
K = 80
M_BLK = 1000
GRID = N // M_BLK
BN_EPS = 1e-5


def _sc_segment_sum(t0, t1, pck, zeros, mode):
    if mode == 'col':
        epw = E // NS
    else:
        epw = E // (NS * NC)
    k = K
    chunks = epw // k
    half = E // (NS * NC)
    half_chunks = half // k
    rows_per_sub = 632

    mesh = plsc.VectorSubcoreMesh(core_axis_name="c", subcore_axis_name="s")

    @functools.partial(
        pl.kernel,
        out_type=(jax.ShapeDtypeStruct((N, 128), jnp.float32),
                  jax.ShapeDtypeStruct((N, 128), jnp.float32)),
        mesh=mesh,
        scratch_types=[
            pltpu.VMEM((E // (NS * NC),), jnp.int32),
            pltpu.VMEM((k,), jnp.int32),
            pltpu.VMEM((k,), jnp.int32),
            pltpu.VMEM((k,), jnp.int32),
            pltpu.VMEM((k,), jnp.int32),
            pltpu.VMEM((k,), jnp.int32),
            pltpu.VMEM((k,), jnp.int32),
            pltpu.VMEM((k, 128), jnp.float32),
            pltpu.VMEM((k, 128), jnp.float32),
            pltpu.VMEM((k, 128), jnp.float32),
            pltpu.VMEM_SHARED((N, 128), jnp.float32),
            pltpu.SemaphoreType.DMA,
            pltpu.SemaphoreType.DMA,
            pltpu.SemaphoreType.DMA,
            pltpu.SemaphoreType.DMA,
            pltpu.SemaphoreType.DMA,
            pltpu.SemaphoreType.DMA,
        ],
    )
    def kern(t0_h, t1_h, pck_h, zero_h, out0_h, out1_h,
             pck_all, sv0, sv1, sv2, dv0, dv1, dv2,
             rv0, rv1, rv2, acc,
             gs0, gs1, gs2, ss0, ss1, ss2):
        bufs = ((sv0, dv0, rv0, gs0, ss0), (sv1, dv1, rv1, gs1, ss1),
                (sv2, dv2, rv2, gs2, ss2))
        c = lax.axis_index("c")
        s = lax.axis_index("s")
        r0 = jnp.minimum(s * rows_per_sub, N - rows_per_sub)
        pltpu.sync_copy(zero_h.at[pl.ds(r0, rows_per_sub)],
                        acc.at[pl.ds(r0, rows_per_sub)])
        plsc.subcore_barrier()

        def run(table_h, out_h, base0):
            pltpu.sync_copy(pck_h.at[pl.ds(base0, half)], pck_all)

            def prefetch(i, buf):
                sv, dv, rv, gs, ss = buf
                if mode == 'col':
                    @pl.when(i == half_chunks)
                    def _():
                        pltpu.sync_copy(pck_h.at[pl.ds(base0 + half, half)],
                                        pck_all)
                    off = i * k - jnp.where(i >= half_chunks, half, 0)
                else:
                    off = i * k
                for j in range(k // 16):
                    v = pck_all[pl.ds(off + j * 16, 16)]
                    sv[pl.ds(j * 16, 16)] = lax.shift_right_logical(v, 14)
                    dv[pl.ds(j * 16, 16)] = lax.bitwise_and(v, 16383)
                pltpu.async_copy(table_h.at[sv], rv, gs)

            prefetch(0, bufs[0])
            prefetch(1, bufs[1])

            @pl.loop(0, ((chunks + 2) // 3) * 3, step=3)
            def _(g):
                for b in (0, 1, 2):
                    sv, dv, rv, gs, ss = bufs[b]
                    i = g + b

                    @pl.when(i < chunks)
                    def _():
                        pltpu.make_async_copy(table_h.at[sv], rv, gs).wait()
                        pltpu.async_copy(rv, acc.at[dv], ss, add=True)

                        @pl.when(i >= 1)
                        def _():
                            svp, dvp, rvp, gsp, ssp = bufs[(b + 2) % 3]
                            pltpu.make_async_copy(rvp, acc.at[dvp], ssp).wait()

                        @pl.when(i + 2 < chunks)
                        def _():
                            prefetch(i + 2, bufs[(b + 2) % 3])

            for j in (chunks - 1,):
                sv, dv, rv, gs, ss = bufs[j % 3]
                pltpu.make_async_copy(rv, acc.at[dv], ss).wait()

            plsc.subcore_barrier()
            pltpu.sync_copy(acc.at[pl.ds(r0, rows_per_sub)],
                            out_h.at[pl.ds(r0, rows_per_sub)])

        if mode == 'col':
            base0 = s * epw
            base1 = s * epw
        else:
            base0 = (s * NC + 0) * epw
            base1 = (s * NC + 1) * epw

        @pl.when(c == 0)
        def _():
            run(t0_h, out0_h, base0)

        @pl.when(c == 1)
        def _():
            run(t1_h, out1_h, base1)

    return kern(t0, t1, pck, zeros)


def _row_spec(cols):
    return pl.BlockSpec((M_BLK, cols), lambda i: (i, 0))


def _fix_spec(shape):
    return pl.BlockSpec(shape, lambda i: (0, 0))


def _emit_stats(i, h, st_ref):
    @pl.when(i == 0)
    def _():
        st_ref[...] = jnp.zeros_like(st_ref)
    st_ref[0:1, :] = st_ref[0:1, :] + jnp.sum(h, axis=0, keepdims=True)
    st_ref[1:2, :] = st_ref[1:2, :] + jnp.sum(h * h, axis=0, keepdims=True)


def _normalize(h, st_ref, g, be, relu):
    mu = st_ref[0:1, :] * (1.0 / N)
    var = st_ref[1:2, :] * (1.0 / N) - mu * mu
    out = g * (h - mu) * lax.rsqrt(var + BN_EPS) + be
    if relu:
        out = jnp.maximum(out, 0.0)
    return out


def _p1_first(x, p0, p1, scale, W1, b1):
    def kern(scale_ref, x_ref, p0_ref, p1_ref, W_ref, b_ref, h_ref, st_ref):
        i = pl.program_id(0)
        z = scale_ref[0, 0] * x_ref[...] + p0_ref[...] + p1_ref[...]
        h = jnp.dot(z, W_ref[...], preferred_element_type=jnp.float32) + b_ref[...]
        h_ref[...] = h
        _emit_stats(i, h, st_ref)

    return pl.pallas_call(
        kern,
        grid=(GRID,),
        in_specs=[pl.BlockSpec(memory_space=pltpu.SMEM),
                  _row_spec(128), _row_spec(128), _row_spec(128),
                  _fix_spec((128, 256)), _fix_spec((1, 256))],
        out_specs=[_row_spec(256), _fix_spec((8, 256))],
        out_shape=[jax.ShapeDtypeStruct((N, 256), jnp.float32),
                   jax.ShapeDtypeStruct((8, 256), jnp.float32)],
    )(scale, x, p0, p1, W1, b1)


def _p1_split(h0, h1c, a0, a1, scale, W1, b1):
    def kern(scale_ref, h0_ref, h1_ref, a0_ref, a1_ref, W_ref, b_ref,
             h_ref, st_ref):
        i = pl.program_id(0)
        sc = scale_ref[0, 0]
        z0 = sc * h0_ref[...] + a0_ref[...]
        z1 = sc * h1_ref[...] + a1_ref[...]
        h = (jnp.dot(z0, W_ref[0:128, :], preferred_element_type=jnp.float32)
             + jnp.dot(z1, W_ref[128:256, :], preferred_element_type=jnp.float32)
             + b_ref[...])
        h_ref[...] = h
        _emit_stats(i, h, st_ref)

    return pl.pallas_call(
        kern,
        grid=(GRID,),
        in_specs=[pl.BlockSpec(memory_space=pltpu.SMEM),
                  _row_spec(128), _row_spec(128), _row_spec(128), _row_spec(128),
                  _fix_spec((256, 256)), _fix_spec((1, 256))],
        out_specs=[_row_spec(256), _fix_spec((8, 256))],
        out_shape=[jax.ShapeDtypeStruct((N, 256), jnp.float32),
                   jax.ShapeDtypeStruct((8, 256), jnp.float32)],
    )(scale, h0, h1c, a0, a1, W1, b1)


def _norm_mm(h_in, st, g, be, W, b):
    def kern(h_ref, st_in_ref, g_ref, be_ref, W_ref, b_ref, h_ref_o, st_ref):
        i = pl.program_id(0)
        a = _normalize(h_ref[...], st_in_ref, g_ref[...], be_ref[...], relu=True)
        h = jnp.dot(a, W_ref[...], preferred_element_type=jnp.float32) + b_ref[...]
        h_ref_o[...] = h
        _emit_stats(i, h, st_ref)

    return pl.pallas_call(
        kern,
        grid=(GRID,),
        in_specs=[_row_spec(256), _fix_spec((8, 256)),
                  _fix_spec((1, 256)), _fix_spec((1, 256)),
                  _fix_spec((256, 256)), _fix_spec((1, 256))],
        out_specs=[_row_spec(256), _fix_spec((8, 256))],
        out_shape=[jax.ShapeDtypeStruct((N, 256), jnp.float32),
                   jax.ShapeDtypeStruct((8, 256), jnp.float32)],
    )(h_in, st, g, be, W, b)


def _norm_split(h_in, st, g, b):
    def kern(h_ref, st_in_ref, g_ref, b_ref, o0_ref, o1_ref):
        h = _normalize(h_ref[...], st_in_ref, g_ref[...], b_ref[...], relu=False)
        o0_ref[...] = h[:, 0:128]
        o1_ref[...] = h[:, 128:256]

    return pl.pallas_call(
        kern,
        grid=(GRID,),
        in_specs=[_row_spec(256), _fix_spec((8, 256)),
                  _fix_spec((1, 256)), _fix_spec((1, 256))],
        out_specs=[_row_spec(128), _row_spec(128)],
        out_shape=[jax.ShapeDtypeStruct((N, 128), jnp.float32),
                   jax.ShapeDtypeStruct((N, 128), jnp.float32)],
    )(h_in, st, g, b)


def _head(ha0, ha1, hb0, hb1, hc, Wf1, bf1, Wf2, bf2):
    def kern(ha0_ref, ha1_ref, hb0_ref, hb1_ref, hc_ref, W1_ref, b1_ref,
             W2_ref, b2_ref, o_ref):
        t = (jnp.dot(ha0_ref[...], W1_ref[0:128, :], preferred_element_type=jnp.float32)
             + jnp.dot(ha1_ref[...], W1_ref[128:256, :], preferred_element_type=jnp.float32)
             + jnp.dot(hb0_ref[...], W1_ref[256:384, :], preferred_element_type=jnp.float32)
             + jnp.dot(hb1_ref[...], W1_ref[384:512, :], preferred_element_type=jnp.float32)
             + jnp.dot(hc_ref[...], W1_ref[512:768, :], preferred_element_type=jnp.float32)
             + b1_ref[...])
        t = jnp.maximum(t, 0.0)
        o_ref[...] = jnp.dot(t, W2_ref[...], preferred_element_type=jnp.float32) + b2_ref[...]

    return pl.pallas_call(
        kern,
        grid=(GRID,),
        in_specs=[_row_spec(128), _row_spec(128), _row_spec(128), _row_spec(128),
                  _row_spec(256),
                  _fix_spec((768, 256)), _fix_spec((1, 256)),
                  _fix_spec((256, 128)), _fix_spec((1, 128))],
        out_specs=pl.BlockSpec((M_BLK, 128), lambda i: (i, 0)),
        out_shape=jax.ShapeDtypeStruct((N, 128), jnp.float32),
    )(ha0, ha1, hb0, hb1, hc, Wf1, bf1, Wf2, bf2)


def kernel(x, edge_index, params):
    src = edge_index[0].astype(jnp.int32)
    dst = edge_index[1].astype(jnp.int32)
    pck = src * 16384 + dst
    zeros = jnp.zeros((N, 128), jnp.float32)
    row = lambda v: v.reshape(1, -1)

    convs = params['convs']
    bns = params['bns']

    def scale_of(p):
        return (1.0 + p['eps']).reshape(1, 1)

    p0, p1 = _sc_segment_sum(x, x, pck, zeros, mode='edge')
    c0 = convs[0]
    h, st = _p1_first(x, p0, p1, scale_of(c0), c0['W1'], row(c0['b1']))
    h, st = _norm_mm(h, st, row(c0['g1']), row(c0['be1']), c0['W2'], row(c0['b2']))
    h, st = _norm_mm(h, st, row(c0['g2']), row(c0['be2']), c0['W3'], row(c0['b3']))
    ha0, ha1 = _norm_split(h, st, row(bns[0]['g']), row(bns[0]['b']))

    a0, a1 = _sc_segment_sum(ha0, ha1, pck, zeros, mode='col')
    c1 = convs[1]
    h, st = _p1_split(ha0, ha1, a0, a1, scale_of(c1), c1['W1'], row(c1['b1']))
    h, st = _norm_mm(h, st, row(c1['g1']), row(c1['be1']), c1['W2'], row(c1['b2']))
    h, st = _norm_mm(h, st, row(c1['g2']), row(c1['be2']), c1['W3'], row(c1['b3']))
    hb0, hb1 = _norm_split(h, st, row(bns[1]['g']), row(bns[1]['b']))

    a0, a1 = _sc_segment_sum(hb0, hb1, pck, zeros, mode='col')
    c2 = convs[2]
    h, st = _p1_split(hb0, hb1, a0, a1, scale_of(c2), c2['W1'], row(c2['b1']))
    h, st = _norm_mm(h, st, row(c2['g1']), row(c2['be1']), c2['W2'], row(c2['b2']))
    hc, _ = _norm_mm(h, st, row(c2['g2']), row(c2['be2']), c2['W3'], row(c2['b3']))

    return _head(ha0, ha1, hb0, hb1, hc,
                 params['fc1']['W'], row(params['fc1']['b']),
                 params['fc2']['W'], row(params['fc2']['b']))


# speedup vs baseline: 9.4874x; 1.0216x over previous
import functools

import jax
import jax.numpy as jnp
from jax import lax
from jax.experimental import pallas as pl
from jax.experimental.pallas import tpu as pltpu
from jax.experimental.pallas import tpu_sc as plsc

N = 10000
E = 320000
NC = 2
NS = 16
K = 80
M_BLK = 2000
GRID = N // M_BLK
BN_EPS = 1e-5


def _sc_segment_sum(t0, t1, pck, zeros, mode):
    if mode == 'col':
        epw = E // NS
    else:
        epw = E // (NS * NC)
    k = K
    chunks = epw // k
    half = E // (NS * NC)
    half_chunks = half // k
    rows_per_sub = 632

    mesh = plsc.VectorSubcoreMesh(core_axis_name="c", subcore_axis_name="s")

    @functools.partial(
        pl.kernel,
        out_type=(jax.ShapeDtypeStruct((N, 128), jnp.float32),
                  jax.ShapeDtypeStruct((N, 128), jnp.float32)),
        mesh=mesh,
        scratch_types=[
            pltpu.VMEM((E // (NS * NC),), jnp.int32),
            pltpu.VMEM((k,), jnp.int32),
            pltpu.VMEM((k,), jnp.int32),
            pltpu.VMEM((k,), jnp.int32),
            pltpu.VMEM((k,), jnp.int32),
            pltpu.VMEM((k,), jnp.int32),
            pltpu.VMEM((k,), jnp.int32),
            pltpu.VMEM((k, 128), jnp.float32),
            pltpu.VMEM((k, 128), jnp.float32),
            pltpu.VMEM((k, 128), jnp.float32),
            pltpu.VMEM_SHARED((N, 128), jnp.float32),
            pltpu.SemaphoreType.DMA,
            pltpu.SemaphoreType.DMA,
            pltpu.SemaphoreType.DMA,
            pltpu.SemaphoreType.DMA,
            pltpu.SemaphoreType.DMA,
            pltpu.SemaphoreType.DMA,
        ],
    )
    def kern(t0_h, t1_h, pck_h, zero_h, out0_h, out1_h,
             pck_all, sv0, sv1, sv2, dv0, dv1, dv2,
             rv0, rv1, rv2, acc,
             gs0, gs1, gs2, ss0, ss1, ss2):
        bufs = ((sv0, dv0, rv0, gs0, ss0), (sv1, dv1, rv1, gs1, ss1),
                (sv2, dv2, rv2, gs2, ss2))
        c = lax.axis_index("c")
        s = lax.axis_index("s")
        r0 = jnp.minimum(s * rows_per_sub, N - rows_per_sub)
        pltpu.sync_copy(zero_h.at[pl.ds(r0, rows_per_sub)],
                        acc.at[pl.ds(r0, rows_per_sub)])
        plsc.subcore_barrier()

        def run(table_h, out_h, base0):
            pltpu.sync_copy(pck_h.at[pl.ds(base0, half)], pck_all)

            def prefetch(i, buf):
                sv, dv, rv, gs, ss = buf
                if mode == 'col':
                    @pl.when(i == half_chunks)
                    def _():
                        pltpu.sync_copy(pck_h.at[pl.ds(base0 + half, half)],
                                        pck_all)
                    off = i * k - jnp.where(i >= half_chunks, half, 0)
                else:
                    off = i * k
                for j in range(k // 16):
                    v = pck_all[pl.ds(off + j * 16, 16)]
                    sv[pl.ds(j * 16, 16)] = lax.shift_right_logical(v, 14)
                    dv[pl.ds(j * 16, 16)] = lax.bitwise_and(v, 16383)
                pltpu.async_copy(table_h.at[sv], rv, gs)

            prefetch(0, bufs[0])
            prefetch(1, bufs[1])

            @pl.loop(0, ((chunks + 2) // 3) * 3, step=3)
            def _(g):
                for b in (0, 1, 2):
                    sv, dv, rv, gs, ss = bufs[b]
                    i = g + b

                    @pl.when(i < chunks)
                    def _():
                        pltpu.make_async_copy(table_h.at[sv], rv, gs).wait()
                        pltpu.async_copy(rv, acc.at[dv], ss, add=True)

                        @pl.when(i >= 1)
                        def _():
                            svp, dvp, rvp, gsp, ssp = bufs[(b + 2) % 3]
                            pltpu.make_async_copy(rvp, acc.at[dvp], ssp).wait()

                        @pl.when(i + 2 < chunks)
                        def _():
                            prefetch(i + 2, bufs[(b + 2) % 3])

            for j in (chunks - 1,):
                sv, dv, rv, gs, ss = bufs[j % 3]
                pltpu.make_async_copy(rv, acc.at[dv], ss).wait()

            plsc.subcore_barrier()
            pltpu.sync_copy(acc.at[pl.ds(r0, rows_per_sub)],
                            out_h.at[pl.ds(r0, rows_per_sub)])

        if mode == 'col':
            base0 = s * epw
            base1 = s * epw
        else:
            base0 = (s * NC + 0) * epw
            base1 = (s * NC + 1) * epw

        @pl.when(c == 0)
        def _():
            run(t0_h, out0_h, base0)

        @pl.when(c == 1)
        def _():
            run(t1_h, out1_h, base1)

    return kern(t0, t1, pck, zeros)


def _row_spec(cols):
    return pl.BlockSpec((M_BLK, cols), lambda i: (i, 0))


def _fix_spec(shape):
    return pl.BlockSpec(shape, lambda i: (0, 0))


def _emit_stats(i, h, st_ref):
    @pl.when(i == 0)
    def _():
        st_ref[...] = jnp.zeros_like(st_ref)
    st_ref[0:1, :] = st_ref[0:1, :] + jnp.sum(h, axis=0, keepdims=True)
    st_ref[1:2, :] = st_ref[1:2, :] + jnp.sum(h * h, axis=0, keepdims=True)


def _normalize(h, st_ref, g, be, relu):
    mu = st_ref[0:1, :] * (1.0 / N)
    var = st_ref[1:2, :] * (1.0 / N) - mu * mu
    out = g * (h - mu) * lax.rsqrt(var + BN_EPS) + be
    if relu:
        out = jnp.maximum(out, 0.0)
    return out


def _p1_first(x, p0, p1, scale, W1, b1):
    def kern(scale_ref, x_ref, p0_ref, p1_ref, W_ref, b_ref, h_ref, st_ref):
        i = pl.program_id(0)
        z = scale_ref[0, 0] * x_ref[...] + p0_ref[...] + p1_ref[...]
        h = jnp.dot(z, W_ref[...], preferred_element_type=jnp.float32) + b_ref[...]
        h_ref[...] = h
        _emit_stats(i, h, st_ref)

    return pl.pallas_call(
        kern,
        grid=(GRID,),
        in_specs=[pl.BlockSpec(memory_space=pltpu.SMEM),
                  _row_spec(128), _row_spec(128), _row_spec(128),
                  _fix_spec((128, 256)), _fix_spec((1, 256))],
        out_specs=[_row_spec(256), _fix_spec((8, 256))],
        out_shape=[jax.ShapeDtypeStruct((N, 256), jnp.float32),
                   jax.ShapeDtypeStruct((8, 256), jnp.float32)],
    )(scale, x, p0, p1, W1, b1)


def _p1_split(h0, h1c, a0, a1, scale, W1, b1):
    def kern(scale_ref, h0_ref, h1_ref, a0_ref, a1_ref, W_ref, b_ref,
             h_ref, st_ref):
        i = pl.program_id(0)
        sc = scale_ref[0, 0]
        z0 = sc * h0_ref[...] + a0_ref[...]
        z1 = sc * h1_ref[...] + a1_ref[...]
        h = (jnp.dot(z0, W_ref[0:128, :], preferred_element_type=jnp.float32)
             + jnp.dot(z1, W_ref[128:256, :], preferred_element_type=jnp.float32)
             + b_ref[...])
        h_ref[...] = h
        _emit_stats(i, h, st_ref)

    return pl.pallas_call(
        kern,
        grid=(GRID,),
        in_specs=[pl.BlockSpec(memory_space=pltpu.SMEM),
                  _row_spec(128), _row_spec(128), _row_spec(128), _row_spec(128),
                  _fix_spec((256, 256)), _fix_spec((1, 256))],
        out_specs=[_row_spec(256), _fix_spec((8, 256))],
        out_shape=[jax.ShapeDtypeStruct((N, 256), jnp.float32),
                   jax.ShapeDtypeStruct((8, 256), jnp.float32)],
    )(scale, h0, h1c, a0, a1, W1, b1)


def _norm_mm(h_in, st, g, be, W, b):
    def kern(h_ref, st_in_ref, g_ref, be_ref, W_ref, b_ref, h_ref_o, st_ref):
        i = pl.program_id(0)
        a = _normalize(h_ref[...], st_in_ref, g_ref[...], be_ref[...], relu=True)
        h = jnp.dot(a, W_ref[...], preferred_element_type=jnp.float32) + b_ref[...]
        h_ref_o[...] = h
        _emit_stats(i, h, st_ref)

    return pl.pallas_call(
        kern,
        grid=(GRID,),
        in_specs=[_row_spec(256), _fix_spec((8, 256)),
                  _fix_spec((1, 256)), _fix_spec((1, 256)),
                  _fix_spec((256, 256)), _fix_spec((1, 256))],
        out_specs=[_row_spec(256), _fix_spec((8, 256))],
        out_shape=[jax.ShapeDtypeStruct((N, 256), jnp.float32),
                   jax.ShapeDtypeStruct((8, 256), jnp.float32)],
    )(h_in, st, g, be, W, b)


def _norm_split(h_in, st, g, b):
    def kern(h_ref, st_in_ref, g_ref, b_ref, o0_ref, o1_ref):
        h = _normalize(h_ref[...], st_in_ref, g_ref[...], b_ref[...], relu=False)
        o0_ref[...] = h[:, 0:128]
        o1_ref[...] = h[:, 128:256]

    return pl.pallas_call(
        kern,
        grid=(GRID,),
        in_specs=[_row_spec(256), _fix_spec((8, 256)),
                  _fix_spec((1, 256)), _fix_spec((1, 256))],
        out_specs=[_row_spec(128), _row_spec(128)],
        out_shape=[jax.ShapeDtypeStruct((N, 128), jnp.float32),
                   jax.ShapeDtypeStruct((N, 128), jnp.float32)],
    )(h_in, st, g, b)


def _head(ha0, ha1, hb0, hb1, hc, Wf1, bf1, Wf2, bf2):
    def kern(ha0_ref, ha1_ref, hb0_ref, hb1_ref, hc_ref, W1_ref, b1_ref,
             W2_ref, b2_ref, o_ref):
        t = (jnp.dot(ha0_ref[...], W1_ref[0:128, :], preferred_element_type=jnp.float32)
             + jnp.dot(ha1_ref[...], W1_ref[128:256, :], preferred_element_type=jnp.float32)
             + jnp.dot(hb0_ref[...], W1_ref[256:384, :], preferred_element_type=jnp.float32)
             + jnp.dot(hb1_ref[...], W1_ref[384:512, :], preferred_element_type=jnp.float32)
             + jnp.dot(hc_ref[...], W1_ref[512:768, :], preferred_element_type=jnp.float32)
             + b1_ref[...])
        t = jnp.maximum(t, 0.0)
        o_ref[...] = jnp.dot(t, W2_ref[...], preferred_element_type=jnp.float32) + b2_ref[...]

    return pl.pallas_call(
        kern,
        grid=(GRID,),
        in_specs=[_row_spec(128), _row_spec(128), _row_spec(128), _row_spec(128),
                  _row_spec(256),
                  _fix_spec((768, 256)), _fix_spec((1, 256)),
                  _fix_spec((256, 128)), _fix_spec((1, 128))],
        out_specs=pl.BlockSpec((M_BLK, 128), lambda i: (i, 0)),
        out_shape=jax.ShapeDtypeStruct((N, 128), jnp.float32),
    )(ha0, ha1, hb0, hb1, hc, Wf1, bf1, Wf2, bf2)


def kernel(x, edge_index, params):
    ei = edge_index.astype(jnp.int32).reshape(2, E // 128, 128)
    pck = (ei[0] * 16384 + ei[1]).reshape(E)
    zeros = jnp.zeros((N, 128), jnp.float32)
    row = lambda v: v.reshape(1, -1)

    convs = params['convs']
    bns = params['bns']

    def scale_of(p):
        return (1.0 + p['eps']).reshape(1, 1)

    p0, p1 = _sc_segment_sum(x, x, pck, zeros, mode='edge')
    c0 = convs[0]
    h, st = _p1_first(x, p0, p1, scale_of(c0), c0['W1'], row(c0['b1']))
    h, st = _norm_mm(h, st, row(c0['g1']), row(c0['be1']), c0['W2'], row(c0['b2']))
    h, st = _norm_mm(h, st, row(c0['g2']), row(c0['be2']), c0['W3'], row(c0['b3']))
    ha0, ha1 = _norm_split(h, st, row(bns[0]['g']), row(bns[0]['b']))

    a0, a1 = _sc_segment_sum(ha0, ha1, pck, zeros, mode='col')
    c1 = convs[1]
    h, st = _p1_split(ha0, ha1, a0, a1, scale_of(c1), c1['W1'], row(c1['b1']))
    h, st = _norm_mm(h, st, row(c1['g1']), row(c1['be1']), c1['W2'], row(c1['b2']))
    h, st = _norm_mm(h, st, row(c1['g2']), row(c1['be2']), c1['W3'], row(c1['b3']))
    hb0, hb1 = _norm_split(h, st, row(bns[1]['g']), row(bns[1]['b']))

    a0, a1 = _sc_segment_sum(hb0, hb1, pck, zeros, mode='col')
    c2 = convs[2]
    h, st = _p1_split(hb0, hb1, a0, a1, scale_of(c2), c2['W1'], row(c2['b1']))
    h, st = _norm_mm(h, st, row(c2['g1']), row(c2['be1']), c2['W2'], row(c2['b2']))
    hc, _ = _norm_mm(h, st, row(c2['g2']), row(c2['be2']), c2['W3'], row(c2['b3']))

    return _head(ha0, ha1, hb0, hb1, hc,
                 params['fc1']['W'], row(params['fc1']['b']),
                 params['fc2']['W'], row(params['fc2']['b']))
